# baseline (speedup 1.0000x reference)
import functools

import jax
import jax.numpy as jnp
from jax import lax
from jax.experimental import pallas as pl
from jax.experimental.pallas import tpu as pltpu
from jax.experimental.pallas import tpu_sc as plsc

N = 10000
K = 64
F = 128
R = 16
E = N * K

S = 1
NP = N // S
EP = NP * K

NC = 2
NS = 16
NW = NC * NS
EPW = EP // NW
CH = 80
CPW = EPW // CH
NBUF = 5
assert CPW % NBUF == 0


def _gather_body(x_hbm, nb_hbm, out_hbm, idx_v, rows, sems):
    wid = lax.axis_index("s") * NC + lax.axis_index("c")
    base = wid * EPW
    pltpu.sync_copy(nb_hbm.at[wid], idx_v)
    for b in range(NBUF):
        pltpu.async_copy(x_hbm.at[idx_v.at[b]], rows[b], sems[b])

    def body(kk, carry):
        for b in range(NBUF):
            j = kk * NBUF + b
            pltpu.make_async_copy(x_hbm.at[idx_v.at[j]], rows[b], sems[b]).wait()
            pltpu.sync_copy(rows[b], out_hbm.at[pl.ds(base + j * CH, CH)])

            @pl.when(j + NBUF < CPW)
            def _():
                pltpu.async_copy(x_hbm.at[idx_v.at[j + NBUF]], rows[b], sems[b])

        return carry

    lax.fori_loop(0, CPW // NBUF, body, 0)


def _gather_entry(x_hbm, nb_hbm, out_hbm, idx_v, *bufs):
    rows = bufs[:NBUF]
    sems = bufs[NBUF:]
    _gather_body(x_hbm, nb_hbm, out_hbm, idx_v, rows, sems)


@functools.cache
def _sc_gather_kernel():
    return pl.kernel(
        _gather_entry,
        out_type=jax.ShapeDtypeStruct((EP, F), jnp.float32),
        mesh=plsc.VectorSubcoreMesh(
            core_axis_name="c", subcore_axis_name="s", num_cores=NC, num_subcores=NS
        ),
        scratch_types=[
            pltpu.VMEM((CPW, CH), jnp.int32),
            *[pltpu.VMEM((CH, F), jnp.float32) for _ in range(NBUF)],
            *[pltpu.SemaphoreType.DMA for _ in range(NBUF)],
        ],
    )


TN = 128
GRID = (NP + TN - 1) // TN


_LOG2E = 1.4426950408889634
_LN2 = 0.6931471805599453


def _tc_body(rbft_ref, xjt_ref, w1t_ref, b1c_ref, w2_ref, b2r_ref, out_ref):
    out_ref[...] = jnp.zeros((TN, F), jnp.float32)

    def body(k, carry):
        rk = rbft_ref[k, :, :]
        ht = jnp.dot(w1t_ref[...], rk, preferred_element_type=jnp.float32)
        ht = ht + b1c_ref[...]
        ht = jnp.log2(1.0 + jnp.exp2(ht * _LOG2E)) * _LN2
        wk = lax.dot_general(
            ht, w2_ref[...], (((0,), (0,)), ((), ())),
            preferred_element_type=jnp.float32,
        )
        wk = wk + b2r_ref[...]
        out_ref[...] += xjt_ref[k, :, :] * wk
        return carry

    lax.fori_loop(0, K, body, 0)


def _tc_cfconv(p, rbft, xjt_p, W1t, b1c, W2, b2r):
    off = p * GRID
    return pl.pallas_call(
        _tc_body,
        grid=(GRID,),
        in_specs=[
            pl.BlockSpec((K, R, TN), lambda i: (0, 0, i + off)),
            pl.BlockSpec((K, TN, F), lambda i: (0, i, 0)),
            pl.BlockSpec((F, R), lambda i: (0, 0)),
            pl.BlockSpec((F, 1), lambda i: (0, 0)),
            pl.BlockSpec((F, F), lambda i: (0, 0)),
            pl.BlockSpec((1, F), lambda i: (0, 0)),
        ],
        out_specs=pl.BlockSpec((TN, F), lambda i: (i, 0)),
        out_shape=jax.ShapeDtypeStruct((NP, F), jnp.float32),
    )(rbft, xjt_p, W1t, b1c, W2, b2r)


def kernel(x, rbf, neighbors, W1, b1, W2, b2):
    rbft = rbf.transpose(1, 2, 0)
    nbt = neighbors.astype(jnp.int32).transpose(1, 0)
    nb = (
        nbt.reshape(K, S, NP)
        .transpose(1, 0, 2)
        .reshape(S, NW, CPW, CH)
    )
    W1t = W1.transpose(1, 0)
    b1c = b1.reshape(F, 1)
    b2r = b2.reshape(1, F)
    gather = _sc_gather_kernel()
    xjt = gather(x, nb[0]).reshape(K, NP, F)
    return _tc_cfconv(0, rbft, xjt, W1t, b1c, W2, b2r)


# speedup vs baseline: 3.4231x; 3.4231x over previous
import functools

import jax
import jax.numpy as jnp
from jax import lax
from jax.experimental import pallas as pl
from jax.experimental.pallas import tpu as pltpu
from jax.experimental.pallas import tpu_sc as plsc

N = 10000
K = 64
F = 128
R = 16
E = N * K

NC = 2
NS = 16
NW = NC * NS
EPW = E // NW
CH = 80
CPW = EPW // CH
NBUF = 5
assert CPW % NBUF == 0


def _gather_body(x_hbm, nb_hbm, out_hbm, idx_v, rows, sems):
    wid = lax.axis_index("s") * NC + lax.axis_index("c")
    base = wid * EPW
    pltpu.sync_copy(nb_hbm.at[wid], idx_v)
    for b in range(NBUF):
        pltpu.async_copy(x_hbm.at[idx_v.at[b]], rows[b], sems[b])

    def body(kk, carry):
        for b in range(NBUF):
            j = kk * NBUF + b
            pltpu.make_async_copy(x_hbm.at[idx_v.at[j]], rows[b], sems[b]).wait()
            pltpu.sync_copy(rows[b], out_hbm.at[pl.ds(base + j * CH, CH)])

            @pl.when(j + NBUF < CPW)
            def _():
                pltpu.async_copy(x_hbm.at[idx_v.at[j + NBUF]], rows[b], sems[b])

        return carry

    lax.fori_loop(0, CPW // NBUF, body, 0)


def _gather_entry(x_hbm, nb_hbm, out_hbm, idx_v, *bufs):
    rows = bufs[:NBUF]
    sems = bufs[NBUF:]
    _gather_body(x_hbm, nb_hbm, out_hbm, idx_v, rows, sems)


@functools.cache
def _sc_gather_kernel():
    return pl.kernel(
        _gather_entry,
        out_type=jax.ShapeDtypeStruct((E, F), jnp.float32),
        mesh=plsc.VectorSubcoreMesh(
            core_axis_name="c", subcore_axis_name="s", num_cores=NC, num_subcores=NS
        ),
        scratch_types=[
            pltpu.VMEM((CPW, CH), jnp.int32),
            *[pltpu.VMEM((CH, F), jnp.float32) for _ in range(NBUF)],
            *[pltpu.SemaphoreType.DMA for _ in range(NBUF)],
        ],
    )


TN = 200
GRID = N // TN


_LOG2E = 1.4426950408889634
_LN2 = 0.6931471805599453


def _tc_body(rbf_ref, xj_ref, w1_ref, b1_ref, w2_ref, b2_ref, out_ref):
    rbf2 = rbf_ref[...].reshape(TN * K, R)
    h = jnp.dot(rbf2, w1_ref[...], preferred_element_type=jnp.float32)
    h = h + b1_ref[...]
    h = jnp.log2(1.0 + jnp.exp2(h * _LOG2E)) * _LN2
    w = jnp.dot(h, w2_ref[...], preferred_element_type=jnp.float32)
    w = w + b2_ref[...]
    prod = xj_ref[...] * w
    out_ref[...] = prod.reshape(TN, K, F).sum(axis=1)


def _tc_cfconv(rbf, xj, W1, b1, W2, b2):
    return pl.pallas_call(
        _tc_body,
        grid=(GRID,),
        in_specs=[
            pl.BlockSpec((TN, K, R), lambda i: (i, 0, 0)),
            pl.BlockSpec((TN * K, F), lambda i: (i, 0)),
            pl.BlockSpec((R, F), lambda i: (0, 0)),
            pl.BlockSpec((1, F), lambda i: (0, 0)),
            pl.BlockSpec((F, F), lambda i: (0, 0)),
            pl.BlockSpec((1, F), lambda i: (0, 0)),
        ],
        out_specs=pl.BlockSpec((TN, F), lambda i: (i, 0)),
        out_shape=jax.ShapeDtypeStruct((N, F), jnp.float32),
    )(rbf, xj, W1, b1, W2, b2)


def kernel(x, rbf, neighbors, W1, b1, W2, b2):
    nb = neighbors.astype(jnp.int32).reshape(NW, CPW, CH)
    xj = _sc_gather_kernel()(x, nb)
    return _tc_cfconv(rbf, xj, W1, b1.reshape(1, F), W2, b2.reshape(1, F))
